# baseline (speedup 1.0000x reference)
import functools

import jax
import jax.numpy as jnp
from jax import lax
from jax.experimental import pallas as pl
from jax.experimental.pallas import tpu as pltpu
from jax.experimental.pallas import tpu_sc as plsc

N_USERS = 1000000
N_ITEMS = 1000000
K = 32
BATCH = 16384

NC = 2
NS = 16
NW = NC * NS
B_PER_W = BATCH // NW
CHUNK = 128
N_CHUNKS = B_PER_W // CHUNK
ROWS_PER_TILE = 128 // K
HALF = B_PER_W // 2

_mesh = plsc.VectorSubcoreMesh(core_axis_name="c", subcore_axis_name="s")

_GATHER_DNUMS = lax.GatherDimensionNumbers(
    offset_dims=(), collapsed_slice_dims=(0,), start_index_map=(0,))


def _vperm(x, idx):
    return lax.gather(x, idx[:, None], _GATHER_DNUMS, slice_sizes=(1,),
                      mode=lax.GatherScatterMode.PROMISE_IN_BOUNDS)


@functools.partial(
    pl.kernel,
    out_type=jax.ShapeDtypeStruct((BATCH,), jnp.float32),
    mesh=_mesh,
    scratch_types=[
        pltpu.VMEM((N_CHUNKS, CHUNK), jnp.int32),
        pltpu.VMEM((N_CHUNKS, CHUNK), jnp.int32),
        pltpu.VMEM((N_CHUNKS, CHUNK), jnp.int32),
        pltpu.VMEM((N_CHUNKS, CHUNK), jnp.int32),
        pltpu.VMEM((HALF, 128), jnp.float32),
        pltpu.VMEM((HALF, 128), jnp.float32),
        pltpu.VMEM((B_PER_W,), jnp.float32),
        pltpu.SemaphoreType.DMA,
    ],
)
def _pmf_kernel(uidx_hbm, iidx_hbm, utab_hbm, itab_hbm, out_hbm,
                uidx_v, iidx_v, utix_v, itix_v, urows_v, irows_v,
                out_v, sem):
    wid = lax.axis_index("s") * NC + lax.axis_index("c")
    base = wid * B_PER_W

    for c in range(N_CHUNKS):
        pltpu.sync_copy(uidx_hbm.at[pl.ds(base + c * CHUNK, CHUNK)],
                        uidx_v.at[c])
        pltpu.sync_copy(iidx_hbm.at[pl.ds(base + c * CHUNK, CHUNK)],
                        iidx_v.at[c])
    for c in range(N_CHUNKS):
        for j in range(CHUNK // 16):
            s = pl.ds(j * 16, 16)
            utix_v[c, s] = uidx_v[c, s] >> 2
            itix_v[c, s] = iidx_v[c, s] >> 2

    lane = lax.iota(jnp.int32, 16)
    perms = [lane ^ (1 << sft) for sft in range(4)]

    def lane_sum(x):
        for p in perms:
            x = x + _vperm(x, p)
        return x

    for h in range(2):
        copies = []
        for cc in range(2):
            c = h * 2 + cc
            copies.append(pltpu.async_copy(
                utab_hbm.at[utix_v.at[c]],
                urows_v.at[pl.ds(cc * CHUNK, CHUNK)], sem))
            copies.append(pltpu.async_copy(
                itab_hbm.at[itix_v.at[c]],
                irows_v.at[pl.ds(cc * CHUNK, CHUNK)], sem))
        for cp in copies:
            cp.wait()

        def grp_body(g, _):
            c_loc = h * 2 + g // (CHUNK // 16)
            rr0 = (g % (CHUNK // 16)) * 16
            uqv = (uidx_v[c_loc, pl.ds(rr0, 16)] & 3) * K
            iqv = (iidx_v[c_loc, pl.ds(rr0, 16)] & 3) * K
            acc = jnp.zeros((16,), jnp.float32)
            for r in range(16):
                i = g * 16 + r
                uq = uqv[r]
                iq = iqv[r]
                p = (urows_v[i, pl.ds(uq, 16)] * irows_v[i, pl.ds(iq, 16)]
                     + urows_v[i, pl.ds(uq + 16, 16)]
                     * irows_v[i, pl.ds(iq + 16, 16)])
                acc = jnp.where(lane == r, lane_sum(p), acc)
            out_v[pl.ds(h * HALF + g * 16, 16)] = acc
            return 0

        lax.fori_loop(0, HALF // 16, grp_body, 0)

    pltpu.sync_copy(out_v, out_hbm.at[pl.ds(base, B_PER_W)])


def kernel(users_index, items_index, user_embeddings, item_embeddings):
    return _pmf_kernel(users_index.astype(jnp.int32),
                       items_index.astype(jnp.int32),
                       user_embeddings.reshape(N_USERS // ROWS_PER_TILE, 128),
                       item_embeddings.reshape(N_ITEMS // ROWS_PER_TILE, 128))
